# baseline (speedup 1.0000x reference)
import jax
import jax.numpy as jnp
from jax.experimental import pallas as pl

B, L, D = 16, 4096, 128


def _body(x_ref, m_ref, o_ref):
    m = m_ref[0, 0]
    s = jnp.sum(x_ref[0] * m[:, None], axis=0)
    o_ref[0] = (s / jnp.sum(m))[None, :]


@jax.jit
def kernel(inputs, mask):
    m = mask.astype(jnp.float32).reshape(B, 1, L)
    out = pl.pallas_call(
        _body,
        grid=(B,),
        in_specs=[
            pl.BlockSpec((1, L, D), lambda b: (b, 0, 0)),
            pl.BlockSpec((1, 1, L), lambda b: (b, 0, 0)),
        ],
        out_specs=pl.BlockSpec((1, 1, D), lambda b: (b, 0, 0)),
        out_shape=jax.ShapeDtypeStruct((B, 1, D), jnp.float32),
    )(inputs, m)
    return out.reshape(B, D)


# speedup vs baseline: 1.1499x; 1.1499x over previous
import jax
import jax.numpy as jnp
from jax.experimental import pallas as pl

B, L, D = 16, 4096, 128
NS = 4
BL = L // NS


def _body(x0, x1, x2, x3, m_ref, o_ref):
    m = m_ref[0, 0]
    s = jnp.zeros((1, D), jnp.float32)
    for k, xr in enumerate((x0, x1, x2, x3)):
        mk = m[k * BL:(k + 1) * BL]
        s = s + jnp.dot(mk[None, :], xr[0, 0],
                        preferred_element_type=jnp.float32)
    o_ref[0] = s / jnp.sum(m)


@jax.jit
def kernel(inputs, mask):
    x4 = inputs.reshape(B, NS, BL, D)
    m = mask.astype(jnp.float32).reshape(B, 1, L)
    out = pl.pallas_call(
        _body,
        grid=(B,),
        in_specs=[
            pl.BlockSpec((1, 1, BL, D), lambda b, k=k: (b, k, 0, 0))
            for k in range(NS)
        ] + [
            pl.BlockSpec((1, 1, L), lambda b: (b, 0, 0)),
        ],
        out_specs=pl.BlockSpec((1, 1, D), lambda b: (b, 0, 0)),
        out_shape=jax.ShapeDtypeStruct((B, 1, D), jnp.float32),
    )(x4, x4, x4, x4, m)
    return out.reshape(B, D)


# speedup vs baseline: 1.4887x; 1.2947x over previous
import jax
import jax.numpy as jnp
from jax import lax
from jax.experimental import pallas as pl
from jax.experimental.pallas import tpu as pltpu

B, L, D = 16, 4096, 128
CB = 2
NBUF = 4
NCHUNK = B // CB


def _body(x_hbm, m_ref, o_ref, buf, sem):
    def issue(c):
        pltpu.make_async_copy(
            x_hbm.at[pl.ds(c * CB, CB)], buf.at[c % NBUF], sem.at[c % NBUF]
        ).start()

    for c in range(NBUF):
        issue(c)

    for c in range(NCHUNK):
        i = c % NBUF
        pltpu.make_async_copy(
            x_hbm.at[pl.ds(c * CB, CB)], buf.at[i], sem.at[i]
        ).wait()
        mf = m_ref[pl.ds(c * CB, CB)].astype(jnp.float32)
        s = lax.dot_general(mf, buf[i],
                            (((2,), (1,)), ((0,), (0,))),
                            preferred_element_type=jnp.float32)
        o_ref[pl.ds(c * CB, CB)] = s / jnp.sum(mf, axis=2, keepdims=True)
        if c + NBUF < NCHUNK:
            issue(c + NBUF)


@jax.jit
def kernel(inputs, mask):
    m3 = mask.reshape(B, 1, L)
    out = pl.pallas_call(
        _body,
        in_specs=[
            pl.BlockSpec(memory_space=pltpu.MemorySpace.HBM),
            pl.BlockSpec(memory_space=pltpu.MemorySpace.VMEM),
        ],
        out_specs=pl.BlockSpec(memory_space=pltpu.MemorySpace.VMEM),
        out_shape=jax.ShapeDtypeStruct((B, 1, D), jnp.float32),
        scratch_shapes=[
            pltpu.VMEM((NBUF, CB, L, D), jnp.float32),
            pltpu.SemaphoreType.DMA((NBUF,)),
        ],
    )(inputs, m3)
    return out.reshape(B, D)


# speedup vs baseline: 1.5722x; 1.0561x over previous
import jax
import jax.numpy as jnp
from jax import lax
from jax.experimental import pallas as pl

B, L, D = 16, 4096, 128
GB = 4


def _body(x_ref, m_ref, o_ref):
    m = m_ref[...].astype(jnp.float32)
    s = lax.dot_general(m, x_ref[...],
                        (((2,), (1,)), ((0,), (0,))),
                        preferred_element_type=jnp.float32)
    o_ref[...] = s / jnp.sum(m, axis=2, keepdims=True)


@jax.jit
def kernel(inputs, mask):
    m3 = mask.reshape(B, 1, L)
    out = pl.pallas_call(
        _body,
        grid=(B // GB,),
        in_specs=[
            pl.BlockSpec((GB, L, D), lambda b: (b, 0, 0)),
            pl.BlockSpec((GB, 1, L), lambda b: (b, 0, 0)),
        ],
        out_specs=pl.BlockSpec((GB, 1, D), lambda b: (b, 0, 0)),
        out_shape=jax.ShapeDtypeStruct((B, 1, D), jnp.float32),
    )(inputs, m3)
    return out.reshape(B, D)


# speedup vs baseline: 1.5842x; 1.0076x over previous
import jax
import jax.numpy as jnp
from jax import lax
from jax.experimental import pallas as pl

B, L, D = 16, 4096, 128
GB = 4
HL = L // 2


def _body(x0_ref, x1_ref, m_ref, o_ref):
    m = m_ref[...].astype(jnp.float32)
    s0 = lax.dot_general(m[:, :, :HL], x0_ref[:, 0],
                         (((2,), (1,)), ((0,), (0,))),
                         preferred_element_type=jnp.float32)
    s1 = lax.dot_general(m[:, :, HL:], x1_ref[:, 0],
                         (((2,), (1,)), ((0,), (0,))),
                         preferred_element_type=jnp.float32)
    o_ref[...] = (s0 + s1) / jnp.sum(m, axis=2, keepdims=True)


@jax.jit
def kernel(inputs, mask):
    x4 = inputs.reshape(B, 2, HL, D)
    m3 = mask.reshape(B, 1, L)
    out = pl.pallas_call(
        _body,
        grid=(B // GB,),
        in_specs=[
            pl.BlockSpec((GB, 1, HL, D), lambda b: (b, 0, 0, 0)),
            pl.BlockSpec((GB, 1, HL, D), lambda b: (b, 1, 0, 0)),
            pl.BlockSpec((GB, 1, L), lambda b: (b, 0, 0)),
        ],
        out_specs=pl.BlockSpec((GB, 1, D), lambda b: (b, 0, 0)),
        out_shape=jax.ShapeDtypeStruct((B, 1, D), jnp.float32),
    )(x4, x4, m3)
    return out.reshape(B, D)
